# Initial kernel scaffold; baseline (speedup 1.0000x reference)
import jax
import jax.numpy as jnp
from jax.experimental import pallas as pl


def kernel(pos, attr, edge_index, params):
    raise NotImplementedError("write your pallas kernel here")


# speedup vs baseline: 3.4273x; 3.4273x over previous
import functools
import jax
import jax.numpy as jnp
import numpy as np
from jax.experimental import pallas as pl
from jax.experimental.pallas import tpu as pltpu


def _attn_body(q_ref, k_ref, v_ref, o_ref, *, n_valid, scale):
    q = q_ref[0]
    k = k_ref[0]
    v = v_ref[0]
    s = jax.lax.dot_general(q, k, (((1,), (1,)), ((), ())),
                            preferred_element_type=jnp.float32) * scale
    col = jax.lax.broadcasted_iota(jnp.int32, s.shape, 1)
    s = jnp.where(col < n_valid, s, -1e30)
    m = jnp.max(s, axis=1, keepdims=True)
    p = jnp.exp(s - m)
    den = jnp.sum(p, axis=1, keepdims=True)
    o = jax.lax.dot_general(p, v, (((1,), (0,)), ((), ())),
                            preferred_element_type=jnp.float32)
    o_ref[0] = o / den


def _flash_attention(q, k, v, n_valid, bq=256):
    H, Np, hd = q.shape
    scale = 1.0 / np.sqrt(hd)
    grid = (H, Np // bq)
    return pl.pallas_call(
        functools.partial(_attn_body, n_valid=n_valid, scale=scale),
        grid=grid,
        in_specs=[
            pl.BlockSpec((1, bq, hd), lambda h, i: (h, i, 0)),
            pl.BlockSpec((1, Np, hd), lambda h, i: (h, 0, 0)),
            pl.BlockSpec((1, Np, hd), lambda h, i: (h, 0, 0)),
        ],
        out_specs=pl.BlockSpec((1, bq, hd), lambda h, i: (h, i, 0)),
        out_shape=jax.ShapeDtypeStruct((H, Np, hd), jnp.float32),
    )(q, k, v)


def _lin(x, p):
    y = x @ p['W']
    if 'b' in p:
        y = y + p['b']
    return y


def _layer_norm(x, g, b, eps=1e-5):
    mu = jnp.mean(x, axis=-1, keepdims=True)
    var = jnp.mean((x - mu) ** 2, axis=-1, keepdims=True)
    return (x - mu) / jnp.sqrt(var + eps) * g + b


def _multi_head_conv(x, pos, src, dst, p):
    N = x.shape[0]
    Wsrc = jnp.concatenate([hp['src']['W'] for hp in p['heads']], axis=1)
    Wlin = jnp.concatenate([hp['lin']['W'] for hp in p['heads']], axis=1)
    blin = jnp.concatenate([hp['lin']['b'] for hp in p['heads']], axis=0)
    Wpos = jnp.concatenate([hp['pos']['W'] for hp in p['heads']], axis=1)
    bpos = jnp.concatenate([hp['pos']['b'] for hp in p['heads']], axis=0)

    A = -(x @ Wsrc)
    XV = x @ Wlin + blin

    rng = jnp.max(pos, axis=0) - jnp.min(pos, axis=0)
    dmax = rng @ jnp.abs(Wpos) + bpos
    M = jnp.max(A, axis=0) + dmax

    d3 = pos[dst] - pos[src]
    delta = d3 @ Wpos + bpos
    e = jnp.exp(A[src] + delta - M)
    S = jax.ops.segment_sum(e, dst, num_segments=N)
    T = jax.ops.segment_sum(e * (XV[src] + delta), dst, num_segments=N)
    cat = T / (S + 1e-16)

    h = jax.nn.relu(_lin(cat, p['p1']))
    return _lin(h, p['p2'])


def _pos_attn(x, pos, p, num_heads=4):
    residual = x
    xn = _layer_norm(x, p['g1'], p['b1'])
    pe = _lin(jax.nn.relu(_lin(pos, p['pe1'])), p['pe2'])
    xc = _lin(jnp.concatenate([xn, pe], axis=-1), p['comb'])
    N, dim = xc.shape
    hd = dim // num_heads
    Np = 10240
    q = _lin(xc, p['q']).reshape(N, num_heads, hd).transpose(1, 0, 2)
    k = _lin(xc, p['k']).reshape(N, num_heads, hd).transpose(1, 0, 2)
    v = _lin(xc, p['v']).reshape(N, num_heads, hd).transpose(1, 0, 2)
    pad = ((0, 0), (0, Np - N), (0, 0))
    o = _flash_attention(jnp.pad(q, pad), jnp.pad(k, pad), jnp.pad(v, pad), N)
    o = o[:, :N, :].transpose(1, 0, 2).reshape(N, dim)
    out = _lin(o, p['o'])
    return _layer_norm(out + residual, p['g2'], p['b2'])


def kernel(pos, attr, edge_index, params):
    N = pos.shape[0]
    loop = jnp.arange(N, dtype=edge_index.dtype)
    src = jnp.concatenate([edge_index[0], loop])
    dst = jnp.concatenate([edge_index[1], loop])
    x = _multi_head_conv(attr, pos, src, dst, params['pt1'])
    x = _pos_attn(x, pos, params['attn'], num_heads=4)
    x = _multi_head_conv(x, pos, src, dst, params['pt2'])
    return x


# speedup vs baseline: 5.3300x; 1.5552x over previous
import functools
import jax
import jax.numpy as jnp
import numpy as np
from jax import lax
from jax.experimental import pallas as pl
from jax.experimental.pallas import tpu as pltpu
from jax.experimental.pallas import tpu_sc as plsc

_NS = 16
_NC = 2
_NP = 10240
_EB = 64


def _edge_sc_body(n_rounds, src_h, dst_h, post_h, ax_h, wb_h, m_h,
                  st_out,
                  idx_v, idxo_v, dstv, post_v, dbuf, ax_v, st_v,
                  wb_v, m_v, st_sh, sem):
    c = lax.axis_index("c")
    sid = lax.axis_index("s")
    ep_pad = src_h.shape[0]
    per_tile = ep_pad // _NS
    n_chunks = per_tile // _EB
    rows_per = _NP // _NS
    rows0 = sid * rows_per

    pltpu.sync_copy(post_h, post_v)

    for r in range(n_rounds):
        g = c * n_rounds + r
        pltpu.sync_copy(wb_h.at[pl.ds(g * 4, 4)], wb_v)
        pltpu.sync_copy(m_h.at[pl.ds(g * 64, 64)], m_v)
        def _z_row(i, _):
            for j in range(8):
                st_v[i, pl.ds(j * 16, 16)] = jnp.zeros((16,), jnp.float32)
            return 0
        lax.fori_loop(0, _EB, _z_row, 0)
        for z in range(rows_per // _EB):
            pltpu.sync_copy(st_v, st_sh.at[pl.ds(rows0 + z * _EB, _EB)])
        plsc.subcore_barrier()

        def _chunk(ci, _):
            base = sid * per_tile + ci * _EB
            pltpu.sync_copy(src_h.at[pl.ds(base, _EB)], idx_v)
            pltpu.sync_copy(dst_h.at[pl.ds(base, _EB)], dstv)
            for j in range(_EB // 16):
                sl = pl.ds(j * 16, 16)
                idxo_v[sl] = idx_v[sl] + g * _NP
            pltpu.async_copy(ax_h.at[idxo_v], ax_v, sem).wait()
            for k in range(_EB // 16):
                sl = pl.ds(k * 16, 16)
                s16 = idx_v[sl]
                d16 = dstv[sl]
                for i in range(3):
                    off = jnp.full((16,), i * _NP, jnp.int32)
                    ps = plsc.load_gather(post_v, [s16 + off])
                    pd = plsc.load_gather(post_v, [d16 + off])
                    dbuf[i, sl] = pd - ps

            def _edge(e, _):
                d0 = dbuf[0, pl.ds(e, 16)][0]
                d1 = dbuf[1, pl.ds(e, 16)][0]
                d2 = dbuf[2, pl.ds(e, 16)][0]
                for j in range(4):
                    sl = pl.ds(j * 16, 16)
                    sl2 = pl.ds(64 + j * 16, 16)
                    delta = (d0 * wb_v[0, sl] + d1 * wb_v[1, sl]
                             + d2 * wb_v[2, sl] + wb_v[3, sl])
                    ee = jnp.exp(ax_v[e, sl] + delta - m_v[sl])
                    st_v[e, sl] = ee
                    st_v[e, sl2] = ee * (ax_v[e, sl2] + delta)
                return 0
            lax.fori_loop(0, _EB, _edge, 0)
            pltpu.sync_copy(st_v, st_sh.at[dstv], add=True)
            return 0
        lax.fori_loop(0, n_chunks, _chunk, 0)
        plsc.subcore_barrier()

        out0 = g * _NP + rows0
        pltpu.sync_copy(st_sh.at[pl.ds(rows0, rows_per)], st_out.at[pl.ds(out0, rows_per)])


def _edge_sc(src_p, dst_p, post, ax_flat, wb_flat, m_flat, n_rounds):
    n_g = _NC * n_rounds
    mesh = plsc.VectorSubcoreMesh(core_axis_name="c", subcore_axis_name="s")
    f = pl.kernel(
        functools.partial(_edge_sc_body, n_rounds),
        out_type=jax.ShapeDtypeStruct((n_g * _NP, 128), jnp.float32),
        mesh=mesh,
        compiler_params=pltpu.CompilerParams(needs_layout_passes=False),
        scratch_types=[
            pltpu.VMEM((_EB,), jnp.int32),
            pltpu.VMEM((_EB,), jnp.int32),
            pltpu.VMEM((_EB,), jnp.int32),
            pltpu.VMEM((3 * _NP,), jnp.float32),
            pltpu.VMEM((3, _EB + 16), jnp.float32),
            pltpu.VMEM((_EB, 128), jnp.float32),
            pltpu.VMEM((_EB, 128), jnp.float32),
            pltpu.VMEM((4, 64), jnp.float32),
            pltpu.VMEM((64,), jnp.float32),
            pltpu.VMEM_SHARED((_NP, 128), jnp.float32),
            pltpu.SemaphoreType.DMA,
        ],
    )
    return f(src_p, dst_p, post, ax_flat, wb_flat, m_flat)


def _attn_body(q_ref, k_ref, v_ref, o_ref, *, n_valid, scale):
    q = q_ref[0]
    k = k_ref[0]
    v = v_ref[0]
    s = jax.lax.dot_general(q, k, (((1,), (1,)), ((), ())),
                            preferred_element_type=jnp.float32) * scale
    col = jax.lax.broadcasted_iota(jnp.int32, s.shape, 1)
    s = jnp.where(col < n_valid, s, -1e30)
    m = jnp.max(s, axis=1, keepdims=True)
    p = jnp.exp(s - m)
    den = jnp.sum(p, axis=1, keepdims=True)
    o = jax.lax.dot_general(p, v, (((1,), (0,)), ((), ())),
                            preferred_element_type=jnp.float32)
    o_ref[0] = o / den


def _flash_attention(q, k, v, n_valid, bq=256):
    H, Np, hd = q.shape
    scale = 1.0 / np.sqrt(hd)
    grid = (H, Np // bq)
    return pl.pallas_call(
        functools.partial(_attn_body, n_valid=n_valid, scale=scale),
        grid=grid,
        in_specs=[
            pl.BlockSpec((1, bq, hd), lambda h, i: (h, i, 0)),
            pl.BlockSpec((1, Np, hd), lambda h, i: (h, 0, 0)),
            pl.BlockSpec((1, Np, hd), lambda h, i: (h, 0, 0)),
        ],
        out_specs=pl.BlockSpec((1, bq, hd), lambda h, i: (h, i, 0)),
        out_shape=jax.ShapeDtypeStruct((H, Np, hd), jnp.float32),
    )(q, k, v)


def _lin(x, p):
    y = x @ p['W']
    if 'b' in p:
        y = y + p['b']
    return y


def _layer_norm(x, g, b, eps=1e-5):
    mu = jnp.mean(x, axis=-1, keepdims=True)
    var = jnp.mean((x - mu) ** 2, axis=-1, keepdims=True)
    return (x - mu) / jnp.sqrt(var + eps) * g + b


def _multi_head_conv(x, pos, src, dst, p):
    N = x.shape[0]
    Wsrc = jnp.concatenate([hp['src']['W'] for hp in p['heads']], axis=1)
    Wlin = jnp.concatenate([hp['lin']['W'] for hp in p['heads']], axis=1)
    blin = jnp.concatenate([hp['lin']['b'] for hp in p['heads']], axis=0)
    Wpos = jnp.concatenate([hp['pos']['W'] for hp in p['heads']], axis=1)
    bpos = jnp.concatenate([hp['pos']['b'] for hp in p['heads']], axis=0)

    A = -(x @ Wsrc)
    XV = x @ Wlin + blin

    rng = jnp.max(pos, axis=0) - jnp.min(pos, axis=0)
    dmax = rng @ jnp.abs(Wpos) + bpos
    M = jnp.max(A, axis=0) + dmax

    Hc = Wsrc.shape[1]
    n_g = Hc // 64
    n_rounds = n_g // _NC
    pad_n = ((0, _NP - N), (0, 0))
    a_g = jnp.pad(A, pad_n).reshape(_NP, n_g, 64).transpose(1, 0, 2)
    xv_g = jnp.pad(XV, pad_n).reshape(_NP, n_g, 64).transpose(1, 0, 2)
    ax_flat = jnp.concatenate([a_g, xv_g], axis=-1).reshape(n_g * _NP, 128)
    wb = jnp.concatenate([Wpos, bpos[None]], axis=0)
    wb_flat = wb.reshape(4, n_g, 64).transpose(1, 0, 2).reshape(n_g * 4, 64)
    post = jnp.pad(pos.T, ((0, 0), (0, _NP - N))).reshape(-1)

    ST = _edge_sc(src, dst, post, ax_flat, wb_flat, M, n_rounds)
    ST = ST.reshape(n_g, _NP, 128)[:, :N, :]
    S, T = ST[..., :64], ST[..., 64:]
    cat = (T / (S + 1e-16)).transpose(1, 0, 2).reshape(N, Hc)

    h = jax.nn.relu(_lin(cat, p['p1']))
    return _lin(h, p['p2'])


def _pos_attn(x, pos, p, num_heads=4):
    residual = x
    xn = _layer_norm(x, p['g1'], p['b1'])
    pe = _lin(jax.nn.relu(_lin(pos, p['pe1'])), p['pe2'])
    xc = _lin(jnp.concatenate([xn, pe], axis=-1), p['comb'])
    N, dim = xc.shape
    hd = dim // num_heads
    Np = 10240
    q = _lin(xc, p['q']).reshape(N, num_heads, hd).transpose(1, 0, 2)
    k = _lin(xc, p['k']).reshape(N, num_heads, hd).transpose(1, 0, 2)
    v = _lin(xc, p['v']).reshape(N, num_heads, hd).transpose(1, 0, 2)
    pad = ((0, 0), (0, Np - N), (0, 0))
    o = _flash_attention(jnp.pad(q, pad), jnp.pad(k, pad), jnp.pad(v, pad), N)
    o = o[:, :N, :].transpose(1, 0, 2).reshape(N, dim)
    out = _lin(o, p['o'])
    return _layer_norm(out + residual, p['g2'], p['b2'])


def kernel(pos, attr, edge_index, params):
    N = pos.shape[0]
    loop = jnp.arange(N, dtype=edge_index.dtype)
    E = edge_index.shape[1] + N
    ep_pad = -(-E // (_NS * _EB)) * (_NS * _EB)
    src = jnp.concatenate([edge_index[0], loop,
                           jnp.zeros((ep_pad - E,), edge_index.dtype)])
    dst = jnp.concatenate([edge_index[1], loop,
                           jnp.full((ep_pad - E,), N, edge_index.dtype)])
    x = _multi_head_conv(attr, pos, src, dst, params['pt1'])
    x = _pos_attn(x, pos, params['attn'], num_heads=4)
    x = _multi_head_conv(x, pos, src, dst, params['pt2'])
    return x


# speedup vs baseline: 9.5887x; 1.7990x over previous
import functools
import jax
import jax.numpy as jnp
import numpy as np
from jax import lax
from jax.experimental import pallas as pl
from jax.experimental.pallas import tpu as pltpu
from jax.experimental.pallas import tpu_sc as plsc

_NS = 16
_NC = 2
_NP = 10240
_EB = 64


def _edge_sc_body(n_rounds, src_h, dst_h, ax_h, pd_h,
                  st_out,
                  ids0, ids1, idd0, idd1, dst0, dst1,
                  ax0, ax1, pd0, pd1, st0, st_sh, sem0, sem1):
    c = lax.axis_index("c")
    sid = lax.axis_index("s")
    ep_pad = src_h.shape[0]
    per_tile = ep_pad // _NS
    n_chunks = per_tile // _EB
    rows_per = _NP // _NS
    rows0 = sid * rows_per
    banks = ((ids0, idd0, dst0, ax0, pd0, st0, sem0),
             (ids1, idd1, dst1, ax1, pd1, st0, sem1))

    def _load_idx(chunk, g, bk):
        ids, idd, dstv = bk[0], bk[1], bk[2]
        base = sid * per_tile + chunk * _EB
        pltpu.sync_copy(src_h.at[pl.ds(base, _EB)], ids)
        pltpu.sync_copy(dst_h.at[pl.ds(base, _EB)], dstv)
        for j in range(_EB // 16):
            sl = pl.ds(j * 16, 16)
            ids[sl] = ids[sl] + g * _NP
            idd[sl] = dstv[sl] + g * _NP

    def _issue(bk):
        pltpu.async_copy(ax_h.at[bk[0]], bk[3], bk[6])
        pltpu.async_copy(pd_h.at[bk[1]], bk[4], bk[6])

    def _wait(bk):
        pltpu.make_async_copy(ax_h.at[bk[0]], bk[3], bk[6]).wait()
        pltpu.make_async_copy(pd_h.at[bk[1]], bk[4], bk[6]).wait()

    for r in range(n_rounds):
        g = c * n_rounds + r
        def _z_row(i, _):
            for j in range(8):
                st0[i, pl.ds(j * 16, 16)] = jnp.zeros((16,), jnp.float32)
            return 0
        lax.fori_loop(0, _EB, _z_row, 0)
        for z in range(rows_per // _EB):
            pltpu.sync_copy(st0, st_sh.at[pl.ds(rows0 + z * _EB, _EB)])
        plsc.subcore_barrier()

        _load_idx(0, g, banks[0])
        _issue(banks[0])

        def _pair(ci, _):
            for b in range(2):
                bk = banks[b]
                nb = banks[1 - b]
                chunk = 2 * ci + b

                @pl.when(chunk + 1 < n_chunks)
                def _():
                    _load_idx(chunk + 1, g, nb)
                    _issue(nb)
                _wait(bk)
                ax_v, pd_v, st_v = bk[3], bk[4], bk[5]

                def _row(q, _):
                    for j in range(4):
                        sl = pl.ds(j * 16, 16)
                        sl2 = pl.ds(64 + j * 16, 16)
                        ee = jnp.exp(ax_v[q, sl] + pd_v[q, sl])
                        st_v[q, sl] = ee
                        st_v[q, sl2] = ee * (ax_v[q, sl2] + pd_v[q, sl2])
                    return 0
                lax.fori_loop(0, _EB, _row, 0)
                pltpu.sync_copy(st_v, st_sh.at[bk[2]], add=True)
            return 0
        lax.fori_loop(0, n_chunks // 2, _pair, 0)
        plsc.subcore_barrier()

        out0 = g * _NP + rows0
        pltpu.sync_copy(st_sh.at[pl.ds(rows0, rows_per)], st_out.at[pl.ds(out0, rows_per)])


def _edge_sc(src_p, dst_p, ax_flat, pd_flat, n_rounds):
    n_g = _NC * n_rounds
    mesh = plsc.VectorSubcoreMesh(core_axis_name="c", subcore_axis_name="s")
    f = pl.kernel(
        functools.partial(_edge_sc_body, n_rounds),
        out_type=jax.ShapeDtypeStruct((n_g * _NP, 128), jnp.float32),
        mesh=mesh,
        compiler_params=pltpu.CompilerParams(needs_layout_passes=False),
        scratch_types=(
            [pltpu.VMEM((_EB,), jnp.int32) for _ in range(6)]
            + [pltpu.VMEM((_EB, 128), jnp.float32) for _ in range(5)]
            + [pltpu.VMEM_SHARED((_NP, 128), jnp.float32),
               pltpu.SemaphoreType.DMA, pltpu.SemaphoreType.DMA]
        ),
    )
    return f(src_p, dst_p, ax_flat, pd_flat)


def _attn_body(q_ref, k_ref, v_ref, o_ref, *, n_valid, scale):
    q = q_ref[0]
    k = k_ref[0]
    v = v_ref[0]
    s = jax.lax.dot_general(q, k, (((1,), (1,)), ((), ())),
                            preferred_element_type=jnp.float32) * scale
    col = jax.lax.broadcasted_iota(jnp.int32, s.shape, 1)
    s = jnp.where(col < n_valid, s, -1e30)
    m = jnp.max(s, axis=1, keepdims=True)
    p = jnp.exp(s - m)
    den = jnp.sum(p, axis=1, keepdims=True)
    o = jax.lax.dot_general(p, v, (((1,), (0,)), ((), ())),
                            preferred_element_type=jnp.float32)
    o_ref[0] = o / den


def _flash_attention(q, k, v, n_valid, bq=256):
    H, Np, hd = q.shape
    scale = 1.0 / np.sqrt(hd)
    grid = (H, Np // bq)
    return pl.pallas_call(
        functools.partial(_attn_body, n_valid=n_valid, scale=scale),
        grid=grid,
        in_specs=[
            pl.BlockSpec((1, bq, hd), lambda h, i: (h, i, 0)),
            pl.BlockSpec((1, Np, hd), lambda h, i: (h, 0, 0)),
            pl.BlockSpec((1, Np, hd), lambda h, i: (h, 0, 0)),
        ],
        out_specs=pl.BlockSpec((1, bq, hd), lambda h, i: (h, i, 0)),
        out_shape=jax.ShapeDtypeStruct((H, Np, hd), jnp.float32),
    )(q, k, v)


def _lin(x, p):
    y = x @ p['W']
    if 'b' in p:
        y = y + p['b']
    return y


def _layer_norm(x, g, b, eps=1e-5):
    mu = jnp.mean(x, axis=-1, keepdims=True)
    var = jnp.mean((x - mu) ** 2, axis=-1, keepdims=True)
    return (x - mu) / jnp.sqrt(var + eps) * g + b


def _multi_head_conv(x, pos, src, dst, p):
    N = x.shape[0]
    Wsrc = jnp.concatenate([hp['src']['W'] for hp in p['heads']], axis=1)
    Wlin = jnp.concatenate([hp['lin']['W'] for hp in p['heads']], axis=1)
    blin = jnp.concatenate([hp['lin']['b'] for hp in p['heads']], axis=0)
    Wpos = jnp.concatenate([hp['pos']['W'] for hp in p['heads']], axis=1)
    bpos = jnp.concatenate([hp['pos']['b'] for hp in p['heads']], axis=0)

    P = pos @ Wpos
    A = -(x @ Wsrc) - P
    XV = x @ Wlin + blin - P
    PB = P + bpos
    M = jnp.max(A, axis=0) + jnp.max(PB, axis=0)

    Hc = Wsrc.shape[1]
    n_g = Hc // 64
    n_rounds = n_g // _NC
    pad_n = ((0, _NP - N), (0, 0))
    def _pack(left, right):
        lg = jnp.pad(left, pad_n).reshape(_NP, n_g, 64).transpose(1, 0, 2)
        rg = jnp.pad(right, pad_n).reshape(_NP, n_g, 64).transpose(1, 0, 2)
        return jnp.concatenate([lg, rg], axis=-1).reshape(n_g * _NP, 128)
    ax_flat = _pack(A, XV)
    pd_flat = _pack(PB - M, PB)

    ST = _edge_sc(src, dst, ax_flat, pd_flat, n_rounds)
    ST = ST.reshape(n_g, _NP, 128)[:, :N, :]
    S, T = ST[..., :64], ST[..., 64:]
    cat = (T / (S + 1e-16)).transpose(1, 0, 2).reshape(N, Hc)

    h = jax.nn.relu(_lin(cat, p['p1']))
    return _lin(h, p['p2'])


def _pos_attn(x, pos, p, num_heads=4):
    residual = x
    xn = _layer_norm(x, p['g1'], p['b1'])
    pe = _lin(jax.nn.relu(_lin(pos, p['pe1'])), p['pe2'])
    xc = _lin(jnp.concatenate([xn, pe], axis=-1), p['comb'])
    N, dim = xc.shape
    hd = dim // num_heads
    Np = 10240
    q = _lin(xc, p['q']).reshape(N, num_heads, hd).transpose(1, 0, 2)
    k = _lin(xc, p['k']).reshape(N, num_heads, hd).transpose(1, 0, 2)
    v = _lin(xc, p['v']).reshape(N, num_heads, hd).transpose(1, 0, 2)
    pad = ((0, 0), (0, Np - N), (0, 0))
    o = _flash_attention(jnp.pad(q, pad), jnp.pad(k, pad), jnp.pad(v, pad), N)
    o = o[:, :N, :].transpose(1, 0, 2).reshape(N, dim)
    out = _lin(o, p['o'])
    return _layer_norm(out + residual, p['g2'], p['b2'])


def kernel(pos, attr, edge_index, params):
    N = pos.shape[0]
    loop = jnp.arange(N, dtype=edge_index.dtype)
    E = edge_index.shape[1] + N
    ep_pad = -(-E // (_NS * _EB * 2)) * (_NS * _EB * 2)
    src = jnp.concatenate([edge_index[0], loop,
                           jnp.zeros((ep_pad - E,), edge_index.dtype)])
    dst = jnp.concatenate([edge_index[1], loop,
                           jnp.full((ep_pad - E,), N, edge_index.dtype)])
    x = _multi_head_conv(attr, pos, src, dst, params['pt1'])
    x = _pos_attn(x, pos, params['attn'], num_heads=4)
    x = _multi_head_conv(x, pos, src, dst, params['pt2'])
    return x
